# Initial kernel scaffold; baseline (speedup 1.0000x reference)
import jax
import jax.numpy as jnp
from jax import lax
from jax.experimental import pallas as pl
from jax.experimental.pallas import tpu as pltpu
from jax.experimental.pallas import tpu_sc as plsc

N_NODES = 10000
FEAT = 128
N_EDGES = 320000

NC = 2
NS = 16
NW = NC * NS
EPW = N_EDGES // NW
K = 40
NCHUNK = EPW // K
ROWS_PS = N_NODES // NS
DRAIN = 125
DEGW = 16
BLK = 1000

_mesh = plsc.VectorSubcoreMesh(core_axis_name="c", subcore_axis_name="s")

_ZERO16 = jnp.zeros((16,), jnp.float32)
_ONE16 = jnp.ones((16,), jnp.float32)


def _deg_body(dst_hbm, degp_hbm, dst_v, ones_v, buf_v, acc_sh):
    cid = lax.axis_index("c")
    sid = lax.axis_index("s")
    wid = cid * NS + sid
    pltpu.sync_copy(dst_hbm.at[wid], dst_v)

    def _initrow(i, c):
        ones_v[i, :] = _ONE16
        return c

    lax.fori_loop(0, K, _initrow, None)

    def _zrow(i, c):
        buf_v[i, :] = _ZERO16
        return c

    lax.fori_loop(0, ROWS_PS, _zrow, None)
    pltpu.sync_copy(buf_v, acc_sh.at[pl.ds(sid * ROWS_PS, ROWS_PS)])
    plsc.subcore_barrier()

    def _chunk(j, c):
        pltpu.sync_copy(ones_v, acc_sh.at[dst_v.at[j]], add=True)
        return c

    lax.fori_loop(0, NCHUNK, _chunk, None)
    plsc.subcore_barrier()
    pltpu.sync_copy(acc_sh.at[pl.ds(sid * ROWS_PS, ROWS_PS)], buf_v)
    pltpu.sync_copy(buf_v, degp_hbm.at[cid, pl.ds(sid * ROWS_PS, ROWS_PS)])


_deg_call = pl.kernel(
    _deg_body,
    out_type=jax.ShapeDtypeStruct((NC, N_NODES, DEGW), jnp.float32),
    mesh=_mesh,
    scratch_types=[
        pltpu.VMEM((NCHUNK, K), jnp.int32),
        pltpu.VMEM((K, DEGW), jnp.float32),
        pltpu.VMEM((ROWS_PS, DEGW), jnp.float32),
        pltpu.VMEM_SHARED((N_NODES, DEGW), jnp.float32),
    ],
)


def _edge_body(y_hbm, src_hbm, dst_hbm, part_hbm,
               src_v, dst_v, rows0, rows1, buf_v, acc_sh, sem0, sem1):
    cid = lax.axis_index("c")
    sid = lax.axis_index("s")
    wid = cid * NS + sid
    pltpu.sync_copy(src_hbm.at[wid], src_v)
    pltpu.sync_copy(dst_hbm.at[wid], dst_v)

    def _zrow(i, c):
        for t in range(FEAT // 16):
            buf_v[i, pl.ds(t * 16, 16)] = _ZERO16
        return c

    lax.fori_loop(0, DRAIN, _zrow, None)
    for t in range(ROWS_PS // DRAIN):
        pltpu.sync_copy(buf_v, acc_sh.at[pl.ds(sid * ROWS_PS + t * DRAIN, DRAIN)])
    plsc.subcore_barrier()

    pltpu.async_copy(y_hbm.at[src_v.at[0]], rows0, sem0)
    pltpu.async_copy(y_hbm.at[src_v.at[1]], rows1, sem1)

    def _pair(i, c):
        j = 2 * i
        pltpu.make_async_copy(y_hbm.at[src_v.at[j]], rows0, sem0).wait()
        pltpu.sync_copy(rows0, acc_sh.at[dst_v.at[j]], add=True)

        @pl.when(j + 2 < NCHUNK)
        def _():
            pltpu.async_copy(y_hbm.at[src_v.at[j + 2]], rows0, sem0)

        pltpu.make_async_copy(y_hbm.at[src_v.at[j + 1]], rows1, sem1).wait()
        pltpu.sync_copy(rows1, acc_sh.at[dst_v.at[j + 1]], add=True)

        @pl.when(j + 3 < NCHUNK)
        def _():
            pltpu.async_copy(y_hbm.at[src_v.at[j + 3]], rows1, sem1)

        return c

    lax.fori_loop(0, NCHUNK // 2, _pair, None)
    plsc.subcore_barrier()
    for t in range(ROWS_PS // DRAIN):
        r0 = sid * ROWS_PS + t * DRAIN
        pltpu.sync_copy(acc_sh.at[pl.ds(r0, DRAIN)], buf_v)
        pltpu.sync_copy(buf_v, part_hbm.at[cid, pl.ds(r0, DRAIN)])


_edge_call = pl.kernel(
    _edge_body,
    out_type=jax.ShapeDtypeStruct((NC, N_NODES, FEAT), jnp.float32),
    mesh=_mesh,
    scratch_types=[
        pltpu.VMEM((NCHUNK, K), jnp.int32),
        pltpu.VMEM((NCHUNK, K), jnp.int32),
        pltpu.VMEM((K, FEAT), jnp.float32),
        pltpu.VMEM((K, FEAT), jnp.float32),
        pltpu.VMEM((DRAIN, FEAT), jnp.float32),
        pltpu.VMEM_SHARED((N_NODES, FEAT), jnp.float32),
        pltpu.SemaphoreType.DMA,
        pltpu.SemaphoreType.DMA,
    ],
)


def _tc1_body(x_ref, ie_ref, degp_ref, w_ref, y_ref, dis_ref):
    deg = degp_ref[0, :, 0:1] + degp_ref[1, :, 0:1] + 1.0
    dis = lax.rsqrt(deg)
    h = ie_ref[...] * (1.0 + x_ref[...])
    y = jnp.dot(h, w_ref[...], preferred_element_type=jnp.float32)
    y_ref[...] = y * dis
    dis_ref[...] = jnp.broadcast_to(dis, (BLK, FEAT))


_tc1 = pl.pallas_call(
    _tc1_body,
    grid=(N_NODES // BLK,),
    in_specs=[
        pl.BlockSpec((BLK, FEAT), lambda i: (i, 0)),
        pl.BlockSpec((BLK, FEAT), lambda i: (i, 0)),
        pl.BlockSpec((NC, BLK, DEGW), lambda i: (0, i, 0)),
        pl.BlockSpec((FEAT, FEAT), lambda i: (0, 0)),
    ],
    out_specs=[
        pl.BlockSpec((BLK, FEAT), lambda i: (i, 0)),
        pl.BlockSpec((BLK, FEAT), lambda i: (i, 0)),
    ],
    out_shape=[
        jax.ShapeDtypeStruct((N_NODES, FEAT), jnp.float32),
        jax.ShapeDtypeStruct((N_NODES, FEAT), jnp.float32),
    ],
)


def _tc2_body(p_ref, y_ref, dis_ref, b_ref, w_ref, o_ref):
    dis = dis_ref[...]
    o = (p_ref[0] + p_ref[1] + y_ref[...]) * dis + b_ref[...]
    h = jnp.where(o >= 0, o, 0.01 * o)
    o_ref[...] = jnp.dot(h, w_ref[...], preferred_element_type=jnp.float32) * dis


_tc2 = pl.pallas_call(
    _tc2_body,
    grid=(N_NODES // BLK,),
    in_specs=[
        pl.BlockSpec((NC, BLK, FEAT), lambda i: (0, i, 0)),
        pl.BlockSpec((BLK, FEAT), lambda i: (i, 0)),
        pl.BlockSpec((BLK, FEAT), lambda i: (i, 0)),
        pl.BlockSpec((1, FEAT), lambda i: (0, 0)),
        pl.BlockSpec((FEAT, FEAT), lambda i: (0, 0)),
    ],
    out_specs=pl.BlockSpec((BLK, FEAT), lambda i: (i, 0)),
    out_shape=jax.ShapeDtypeStruct((N_NODES, FEAT), jnp.float32),
)


def _tc3_body(p_ref, y_ref, dis_ref, b_ref, o_ref):
    o = (p_ref[0] + p_ref[1] + y_ref[...]) * dis_ref[...] + b_ref[...]
    o_ref[...] = jnp.where(o >= 0, o, 0.01 * o)


_tc3 = pl.pallas_call(
    _tc3_body,
    grid=(N_NODES // BLK,),
    in_specs=[
        pl.BlockSpec((NC, BLK, FEAT), lambda i: (0, i, 0)),
        pl.BlockSpec((BLK, FEAT), lambda i: (i, 0)),
        pl.BlockSpec((BLK, FEAT), lambda i: (i, 0)),
        pl.BlockSpec((1, FEAT), lambda i: (0, 0)),
    ],
    out_specs=pl.BlockSpec((BLK, FEAT), lambda i: (i, 0)),
    out_shape=jax.ShapeDtypeStruct((N_NODES, FEAT), jnp.float32),
)


def kernel(x, edge_index, initial_embedding, W1, b1, W2, b2):
    src = edge_index[0].astype(jnp.int32).reshape(NW, NCHUNK, K)
    dst = edge_index[1].astype(jnp.int32).reshape(NW, NCHUNK, K)
    degp = _deg_call(dst)
    y0, dis = _tc1(x, initial_embedding, degp, W1)
    p0 = _edge_call(y0, src, dst)
    y1 = _tc2(p0, y0, dis, b1.reshape(1, FEAT), W2)
    p1 = _edge_call(y1, src, dst)
    return _tc3(p1, y1, dis, b2.reshape(1, FEAT))


# speedup vs baseline: 14.2213x; 14.2213x over previous
import jax
import jax.numpy as jnp
from jax import lax
from jax.experimental import pallas as pl
from jax.experimental.pallas import tpu as pltpu
from jax.experimental.pallas import tpu_sc as plsc

N_NODES = 10000
FEAT = 128
HF = FEAT // 2
N_EDGES = 320000

NC = 2
NS = 16
NW = NC * NS
K = 40
NCHUNK_D = N_EDGES // NW // K
NCHUNK_E = N_EDGES // NS // K
N_PAD = 10240
ROWS_PS = N_PAD // NS
DRAIN = 128
DEGW = 16
BLK = 1000

_mesh = plsc.VectorSubcoreMesh(
    core_axis_name="c", subcore_axis_name="s", num_cores=NC, num_subcores=NS
)


def _deg_body(dst_hbm, degp_hbm, dst_v, ones_v, buf_v, acc_sh):
    cid = lax.axis_index("c")
    sid = lax.axis_index("s")
    wid = cid * NS + sid
    pltpu.sync_copy(dst_hbm.at[wid], dst_v)

    def _initrow(i, c):
        ones_v[i, :] = jnp.ones((16,), jnp.float32)
        return c

    lax.fori_loop(0, K, _initrow, None)

    def _zrow(i, c):
        buf_v[i, :] = jnp.zeros((16,), jnp.float32)
        return c

    lax.fori_loop(0, ROWS_PS, _zrow, None)
    pltpu.sync_copy(buf_v, acc_sh.at[pl.ds(sid * ROWS_PS, ROWS_PS)])
    plsc.subcore_barrier()

    def _chunk(j, c):
        pltpu.sync_copy(ones_v, acc_sh.at[dst_v.at[j]], add=True)
        return c

    lax.fori_loop(0, NCHUNK_D, _chunk, None)
    plsc.subcore_barrier()
    pltpu.sync_copy(acc_sh.at[pl.ds(sid * ROWS_PS, ROWS_PS)], buf_v)
    pltpu.sync_copy(buf_v, degp_hbm.at[cid, pl.ds(sid * ROWS_PS, ROWS_PS)])


_deg_call = pl.kernel(
    _deg_body,
    out_type=jax.ShapeDtypeStruct((NC, N_PAD, DEGW), jnp.float32),
    mesh=_mesh,
    scratch_types=[
        pltpu.VMEM((NCHUNK_D, K), jnp.int32),
        pltpu.VMEM((K, DEGW), jnp.float32),
        pltpu.VMEM((ROWS_PS, DEGW), jnp.float32),
        pltpu.VMEM_SHARED((N_PAD, DEGW), jnp.float32),
    ],
    compiler_params=pltpu.CompilerParams(use_tc_tiling_on_sc=False),
)


def _edge_body(y_hbm, src_hbm, dst_hbm, part_hbm,
               src_v, dst_v, rows0, rows1, buf_v, acc_sh, sem0, sem1):
    cid = lax.axis_index("c")
    sid = lax.axis_index("s")
    pltpu.sync_copy(src_hbm.at[cid, sid], src_v)
    pltpu.sync_copy(dst_hbm.at[sid], dst_v)

    def _zrow(i, c):
        for t in range(HF // 16):
            buf_v[i, pl.ds(t * 16, 16)] = jnp.zeros((16,), jnp.float32)
        return c

    lax.fori_loop(0, DRAIN, _zrow, None)
    for t in range(ROWS_PS // DRAIN):
        pltpu.sync_copy(buf_v, acc_sh.at[pl.ds(sid * ROWS_PS + t * DRAIN, DRAIN)])
    plsc.subcore_barrier()

    pltpu.async_copy(y_hbm.at[src_v.at[0]], rows0, sem0)
    pltpu.async_copy(y_hbm.at[src_v.at[1]], rows1, sem1)

    def _pair(i, c):
        j = 2 * i
        pltpu.make_async_copy(y_hbm.at[src_v.at[j]], rows0, sem0).wait()
        pltpu.sync_copy(rows0, acc_sh.at[dst_v.at[j]], add=True)

        @pl.when(j + 2 < NCHUNK_E)
        def _():
            pltpu.async_copy(y_hbm.at[src_v.at[j + 2]], rows0, sem0)

        pltpu.make_async_copy(y_hbm.at[src_v.at[j + 1]], rows1, sem1).wait()
        pltpu.sync_copy(rows1, acc_sh.at[dst_v.at[j + 1]], add=True)

        @pl.when(j + 3 < NCHUNK_E)
        def _():
            pltpu.async_copy(y_hbm.at[src_v.at[j + 3]], rows1, sem1)

        return c

    lax.fori_loop(0, NCHUNK_E // 2, _pair, None)
    plsc.subcore_barrier()
    for t in range(ROWS_PS // DRAIN):
        r0 = sid * ROWS_PS + t * DRAIN
        pltpu.sync_copy(acc_sh.at[pl.ds(r0, DRAIN)], buf_v)
        pltpu.sync_copy(buf_v, part_hbm.at[cid, pl.ds(r0, DRAIN)])


_edge_call = pl.kernel(
    _edge_body,
    out_type=jax.ShapeDtypeStruct((NC, N_PAD, HF), jnp.float32),
    mesh=_mesh,
    scratch_types=[
        pltpu.VMEM((NCHUNK_E, K), jnp.int32),
        pltpu.VMEM((NCHUNK_E, K), jnp.int32),
        pltpu.VMEM((K, HF), jnp.float32),
        pltpu.VMEM((K, HF), jnp.float32),
        pltpu.VMEM((DRAIN, HF), jnp.float32),
        pltpu.VMEM_SHARED((N_PAD, HF), jnp.float32),
        pltpu.SemaphoreType.DMA,
        pltpu.SemaphoreType.DMA,
    ],
    compiler_params=pltpu.CompilerParams(use_tc_tiling_on_sc=False),
)


def _tc_prep_body(x_ref, ie_ref, degp_ref, h_ref, dis_ref):
    deg = degp_ref[0, :, 0:1] + degp_ref[1, :, 0:1] + 1.0
    dis = lax.rsqrt(deg)
    h_ref[...] = ie_ref[...] * (1.0 + x_ref[...])
    dis_ref[...] = jnp.broadcast_to(dis, (BLK, FEAT))


_tc_prep = pl.pallas_call(
    _tc_prep_body,
    grid=(N_NODES // BLK,),
    in_specs=[
        pl.BlockSpec((BLK, FEAT), lambda i: (i, 0)),
        pl.BlockSpec((BLK, FEAT), lambda i: (i, 0)),
        pl.BlockSpec((NC, BLK, DEGW), lambda i: (0, i, 0)),
    ],
    out_specs=[
        pl.BlockSpec((BLK, FEAT), lambda i: (i, 0)),
        pl.BlockSpec((BLK, FEAT), lambda i: (i, 0)),
    ],
    out_shape=[
        jax.ShapeDtypeStruct((N_NODES, FEAT), jnp.float32),
        jax.ShapeDtypeStruct((N_NODES, FEAT), jnp.float32),
    ],
)


def _tc_a_body(h_ref, w_ref, dis_ref, y_ref, ysp_ref):
    y = jnp.dot(h_ref[...], w_ref[...], preferred_element_type=jnp.float32)
    y = y * dis_ref[...]
    y_ref[...] = y
    ysp_ref[0] = y[:, :HF]
    ysp_ref[1] = y[:, HF:]


_tc_a = pl.pallas_call(
    _tc_a_body,
    grid=(N_NODES // BLK,),
    in_specs=[
        pl.BlockSpec((BLK, FEAT), lambda i: (i, 0)),
        pl.BlockSpec((FEAT, FEAT), lambda i: (0, 0)),
        pl.BlockSpec((BLK, FEAT), lambda i: (i, 0)),
    ],
    out_specs=[
        pl.BlockSpec((BLK, FEAT), lambda i: (i, 0)),
        pl.BlockSpec((NC, BLK, HF), lambda i: (0, i, 0)),
    ],
    out_shape=[
        jax.ShapeDtypeStruct((N_NODES, FEAT), jnp.float32),
        jax.ShapeDtypeStruct((NC, N_NODES, HF), jnp.float32),
    ],
)


def _tc_b_body(p_ref, y_ref, dis_ref, b_ref, o_ref):
    agg = jnp.concatenate([p_ref[0], p_ref[1]], axis=-1) + y_ref[...]
    o = agg * dis_ref[...] + b_ref[...]
    o_ref[...] = jnp.where(o >= 0, o, 0.01 * o)


_tc_b = pl.pallas_call(
    _tc_b_body,
    grid=(N_NODES // BLK,),
    in_specs=[
        pl.BlockSpec((NC, BLK, HF), lambda i: (0, i, 0)),
        pl.BlockSpec((BLK, FEAT), lambda i: (i, 0)),
        pl.BlockSpec((BLK, FEAT), lambda i: (i, 0)),
        pl.BlockSpec((1, FEAT), lambda i: (0, 0)),
    ],
    out_specs=pl.BlockSpec((BLK, FEAT), lambda i: (i, 0)),
    out_shape=jax.ShapeDtypeStruct((N_NODES, FEAT), jnp.float32),
)


def kernel(x, edge_index, initial_embedding, W1, b1, W2, b2):
    src = edge_index[0].astype(jnp.int32)
    dst = edge_index[1].astype(jnp.int32)
    dst_d = dst.reshape(NW, NCHUNK_D, K)
    dst_e = dst.reshape(NS, NCHUNK_E, K)
    src_e = (src.reshape(1, NS, NCHUNK_E, K)
             + (jnp.arange(NC, dtype=jnp.int32) * N_NODES)[:, None, None, None])

    degp = _deg_call(dst_d)
    h0, dis = _tc_prep(x, initial_embedding, degp)
    Ws = jnp.stack([W1, W2])
    bs = jnp.stack([b1.reshape(1, FEAT), b2.reshape(1, FEAT)])

    def _step(h, wb):
        w, b = wb
        y, ysp = _tc_a(h, w, dis)
        p = _edge_call(ysp.reshape(NC * N_NODES, HF), src_e, dst_e)
        return _tc_b(p, y, dis, b), None

    h_out, _ = lax.scan(_step, h0, (Ws, bs))
    return h_out


# speedup vs baseline: 22.2379x; 1.5637x over previous
import jax
import jax.numpy as jnp
from jax import lax
from jax.experimental import pallas as pl
from jax.experimental.pallas import tpu as pltpu
from jax.experimental.pallas import tpu_sc as plsc

N_NODES = 10000
FEAT = 128
HF = FEAT // 2
N_EDGES = 320000

NC = 2
NS = 16
NW = NC * NS
K = 40
KE = 80
GRP = 5
NCHUNK_D = N_EDGES // NW // K
NCHUNK_E = N_EDGES // NS // KE
NSEG = 5
SEG = NCHUNK_E // NSEG
N_PAD = 10240
ROWS_PS = N_PAD // NS
DRAIN = 128
DEGW = 16
BLK = 1000

_mesh = plsc.VectorSubcoreMesh(
    core_axis_name="c", subcore_axis_name="s", num_cores=NC, num_subcores=NS
)


def _deg_body(dst_hbm, degp_hbm, dst_v, ones_v, buf_v, acc_sh):
    cid = lax.axis_index("c")
    sid = lax.axis_index("s")
    wid = cid * NS + sid
    pltpu.sync_copy(dst_hbm.at[wid], dst_v)

    def _initrow(i, c):
        ones_v[i, :] = jnp.ones((16,), jnp.float32)
        return c

    lax.fori_loop(0, K, _initrow, None)

    def _zrow(i, c):
        buf_v[i, :] = jnp.zeros((16,), jnp.float32)
        return c

    lax.fori_loop(0, ROWS_PS, _zrow, None)
    pltpu.sync_copy(buf_v, acc_sh.at[pl.ds(sid * ROWS_PS, ROWS_PS)])
    plsc.subcore_barrier()

    def _chunk(j, c):
        pltpu.sync_copy(ones_v, acc_sh.at[dst_v.at[j]], add=True)
        return c

    lax.fori_loop(0, NCHUNK_D, _chunk, None)
    plsc.subcore_barrier()
    pltpu.sync_copy(acc_sh.at[pl.ds(sid * ROWS_PS, ROWS_PS)], buf_v)
    pltpu.sync_copy(buf_v, degp_hbm.at[cid, pl.ds(sid * ROWS_PS, ROWS_PS)])


_deg_call = pl.kernel(
    _deg_body,
    out_type=jax.ShapeDtypeStruct((NC, N_PAD, DEGW), jnp.float32),
    mesh=_mesh,
    scratch_types=[
        pltpu.VMEM((NCHUNK_D, K), jnp.int32),
        pltpu.VMEM((K, DEGW), jnp.float32),
        pltpu.VMEM((ROWS_PS, DEGW), jnp.float32),
        pltpu.VMEM_SHARED((N_PAD, DEGW), jnp.float32),
    ],
    compiler_params=pltpu.CompilerParams(use_tc_tiling_on_sc=False),
)


def _edge_body(y_hbm, src_hbm, dst_hbm, part_hbm,
               src_v, dst_v, rows_a, rows_b, acc_sh, sem_a, sem_b):
    cid = lax.axis_index("c")
    sid = lax.axis_index("s")

    def _zrow(i, c):
        for t in range(HF // 16):
            rows_a[i, pl.ds(t * 16, 16)] = jnp.zeros((16,), jnp.float32)
        return c

    lax.fori_loop(0, DRAIN, _zrow, None)
    for t in range(ROWS_PS // DRAIN):
        pltpu.sync_copy(rows_a.at[pl.ds(0, DRAIN)],
                        acc_sh.at[pl.ds(sid * ROWS_PS + t * DRAIN, DRAIN)])
    plsc.subcore_barrier()

    def _fire(j0, rows, sem):
        for b in range(GRP):
            pltpu.async_copy(
                y_hbm.at[src_v.at[j0 + b]], rows.at[pl.ds(b * KE, KE)], sem)

    def _drain_scatter(j0, rows, sem):
        for b in range(GRP):
            pltpu.make_async_copy(
                y_hbm.at[src_v.at[j0 + b]], rows.at[pl.ds(b * KE, KE)],
                sem).wait()
        for b in range(GRP):
            pltpu.sync_copy(rows.at[pl.ds(b * KE, KE)],
                            acc_sh.at[dst_v.at[j0 + b]], add=True)

    for seg in range(NSEG):
        pltpu.sync_copy(src_hbm.at[cid, sid, pl.ds(seg * SEG, SEG)], src_v)
        pltpu.sync_copy(dst_hbm.at[sid, pl.ds(seg * SEG, SEG)], dst_v)
        _fire(0, rows_a, sem_a)

        def _two_groups(i, c):
            j0 = 2 * GRP * i
            _fire(j0 + GRP, rows_b, sem_b)
            _drain_scatter(j0, rows_a, sem_a)

            @pl.when(j0 + 2 * GRP < SEG)
            def _():
                _fire(j0 + 2 * GRP, rows_a, sem_a)

            _drain_scatter(j0 + GRP, rows_b, sem_b)
            return c

        lax.fori_loop(0, SEG // (2 * GRP), _two_groups, None)

    plsc.subcore_barrier()
    for t in range(ROWS_PS // DRAIN):
        r0 = sid * ROWS_PS + t * DRAIN
        pltpu.sync_copy(acc_sh.at[pl.ds(r0, DRAIN)], rows_a.at[pl.ds(0, DRAIN)])
        pltpu.sync_copy(rows_a.at[pl.ds(0, DRAIN)],
                        part_hbm.at[cid, pl.ds(r0, DRAIN)])


_edge_call = pl.kernel(
    _edge_body,
    out_type=jax.ShapeDtypeStruct((NC, N_PAD, HF), jnp.float32),
    mesh=_mesh,
    scratch_types=[
        pltpu.VMEM((SEG, KE), jnp.int32),
        pltpu.VMEM((SEG, KE), jnp.int32),
        pltpu.VMEM((GRP * KE, HF), jnp.float32),
        pltpu.VMEM((GRP * KE, HF), jnp.float32),
        pltpu.VMEM_SHARED((N_PAD, HF), jnp.float32),
        pltpu.SemaphoreType.DMA,
        pltpu.SemaphoreType.DMA,
    ],
    compiler_params=pltpu.CompilerParams(use_tc_tiling_on_sc=False),
)


def _tc_prep_body(x_ref, ie_ref, degp_ref, h_ref, dis_ref):
    deg = degp_ref[0, :, 0:1] + degp_ref[1, :, 0:1] + 1.0
    dis = lax.rsqrt(deg)
    h_ref[...] = ie_ref[...] * (1.0 + x_ref[...])
    dis_ref[...] = jnp.broadcast_to(dis, (BLK, FEAT))


_tc_prep = pl.pallas_call(
    _tc_prep_body,
    grid=(N_NODES // BLK,),
    in_specs=[
        pl.BlockSpec((BLK, FEAT), lambda i: (i, 0)),
        pl.BlockSpec((BLK, FEAT), lambda i: (i, 0)),
        pl.BlockSpec((NC, BLK, DEGW), lambda i: (0, i, 0)),
    ],
    out_specs=[
        pl.BlockSpec((BLK, FEAT), lambda i: (i, 0)),
        pl.BlockSpec((BLK, FEAT), lambda i: (i, 0)),
    ],
    out_shape=[
        jax.ShapeDtypeStruct((N_NODES, FEAT), jnp.float32),
        jax.ShapeDtypeStruct((N_NODES, FEAT), jnp.float32),
    ],
)


def _tc_a_body(h_ref, w_ref, dis_ref, y_ref, ysp_ref):
    y = jnp.dot(h_ref[...], w_ref[...], preferred_element_type=jnp.float32)
    y = y * dis_ref[...]
    y_ref[...] = y
    ysp_ref[0] = y[:, :HF]
    ysp_ref[1] = y[:, HF:]


_tc_a = pl.pallas_call(
    _tc_a_body,
    grid=(N_NODES // BLK,),
    in_specs=[
        pl.BlockSpec((BLK, FEAT), lambda i: (i, 0)),
        pl.BlockSpec((FEAT, FEAT), lambda i: (0, 0)),
        pl.BlockSpec((BLK, FEAT), lambda i: (i, 0)),
    ],
    out_specs=[
        pl.BlockSpec((BLK, FEAT), lambda i: (i, 0)),
        pl.BlockSpec((NC, BLK, HF), lambda i: (0, i, 0)),
    ],
    out_shape=[
        jax.ShapeDtypeStruct((N_NODES, FEAT), jnp.float32),
        jax.ShapeDtypeStruct((NC, N_NODES, HF), jnp.float32),
    ],
)


def _tc_b_body(p_ref, y_ref, dis_ref, b_ref, o_ref):
    agg = jnp.concatenate([p_ref[0], p_ref[1]], axis=-1) + y_ref[...]
    o = agg * dis_ref[...] + b_ref[...]
    o_ref[...] = jnp.where(o >= 0, o, 0.01 * o)


_tc_b = pl.pallas_call(
    _tc_b_body,
    grid=(N_NODES // BLK,),
    in_specs=[
        pl.BlockSpec((NC, BLK, HF), lambda i: (0, i, 0)),
        pl.BlockSpec((BLK, FEAT), lambda i: (i, 0)),
        pl.BlockSpec((BLK, FEAT), lambda i: (i, 0)),
        pl.BlockSpec((1, FEAT), lambda i: (0, 0)),
    ],
    out_specs=pl.BlockSpec((BLK, FEAT), lambda i: (i, 0)),
    out_shape=jax.ShapeDtypeStruct((N_NODES, FEAT), jnp.float32),
)


def kernel(x, edge_index, initial_embedding, W1, b1, W2, b2):
    src = edge_index[0].astype(jnp.int32)
    dst = edge_index[1].astype(jnp.int32)
    dst_d = dst.reshape(NW, NCHUNK_D, K)
    dst_e = dst.reshape(NS, NCHUNK_E, KE)
    src_e = (src.reshape(1, NS, NCHUNK_E, KE)
             + (jnp.arange(NC, dtype=jnp.int32) * N_NODES)[:, None, None, None])

    degp = _deg_call(dst_d)
    h0, dis = _tc_prep(x, initial_embedding, degp)
    Ws = jnp.stack([W1, W2])
    bs = jnp.stack([b1.reshape(1, FEAT), b2.reshape(1, FEAT)])

    def _step(h, wb):
        w, b = wb
        y, ysp = _tc_a(h, w, dis)
        p = _edge_call(ysp.reshape(NC * N_NODES, HF), src_e, dst_e)
        return _tc_b(p, y, dis, b), None

    h_out, _ = lax.scan(_step, h0, (Ws, bs))
    return h_out


# speedup vs baseline: 25.2375x; 1.1349x over previous
import jax
import jax.numpy as jnp
from jax import lax
from jax.experimental import pallas as pl
from jax.experimental.pallas import tpu as pltpu
from jax.experimental.pallas import tpu_sc as plsc

N_NODES = 10000
FEAT = 128
HF = FEAT // 2
N_EDGES = 320000

NC = 2
NS = 16
NW = NC * NS
K = 40
KE = 80
GRP = 5
NCHUNK_D = N_EDGES // NW // K
NCHUNK_E = N_EDGES // NS // KE
NSEG = 5
SEG = NCHUNK_E // NSEG
N_PAD = 10240
ROWS_PS = N_PAD // NS
DRAIN = 128
DEGW = 16
BLK = 1000

_mesh = plsc.VectorSubcoreMesh(
    core_axis_name="c", subcore_axis_name="s", num_cores=NC, num_subcores=NS
)


def _deg_body(dst_hbm, degp_hbm, dst_v, ones_v, buf_v, acc_sh, sem_d):
    cid = lax.axis_index("c")
    sid = lax.axis_index("s")
    wid = cid * NS + sid
    pltpu.sync_copy(dst_hbm.at[wid], dst_v)

    def _initrow(i, c):
        ones_v[i, :] = jnp.ones((16,), jnp.float32)
        return c

    lax.fori_loop(0, K, _initrow, None)

    def _zrow(i, c):
        buf_v[i, :] = jnp.zeros((16,), jnp.float32)
        return c

    lax.fori_loop(0, ROWS_PS, _zrow, None)
    pltpu.sync_copy(buf_v, acc_sh.at[pl.ds(sid * ROWS_PS, ROWS_PS)])
    plsc.subcore_barrier()

    def _grp(i, c):
        j0 = 10 * i
        for b in range(10):
            pltpu.async_copy(ones_v, acc_sh.at[dst_v.at[j0 + b]], sem_d,
                             add=True)
        for b in range(10):
            pltpu.make_async_copy(ones_v, acc_sh.at[dst_v.at[j0 + b]],
                                  sem_d).wait()
        return c

    lax.fori_loop(0, NCHUNK_D // 10, _grp, None)
    plsc.subcore_barrier()
    pltpu.sync_copy(acc_sh.at[pl.ds(sid * ROWS_PS, ROWS_PS)], buf_v)
    pltpu.sync_copy(buf_v, degp_hbm.at[cid, pl.ds(sid * ROWS_PS, ROWS_PS)])


_deg_call = pl.kernel(
    _deg_body,
    out_type=jax.ShapeDtypeStruct((NC, N_PAD, DEGW), jnp.float32),
    mesh=_mesh,
    scratch_types=[
        pltpu.VMEM((NCHUNK_D, K), jnp.int32),
        pltpu.VMEM((K, DEGW), jnp.float32),
        pltpu.VMEM((ROWS_PS, DEGW), jnp.float32),
        pltpu.VMEM_SHARED((N_PAD, DEGW), jnp.float32),
        pltpu.SemaphoreType.DMA,
    ],
    compiler_params=pltpu.CompilerParams(use_tc_tiling_on_sc=False),
)


def _edge_body(y_hbm, src_hbm, dst_hbm, part_hbm,
               src_v, dst_v, rows_a, rows_b, acc_sh, sem_a, sem_b):
    cid = lax.axis_index("c")
    sid = lax.axis_index("s")

    def _zrow(i, c):
        for t in range(HF // 16):
            rows_a[i, pl.ds(t * 16, 16)] = jnp.zeros((16,), jnp.float32)
        return c

    lax.fori_loop(0, DRAIN, _zrow, None)
    for t in range(ROWS_PS // DRAIN):
        pltpu.sync_copy(rows_a.at[pl.ds(0, DRAIN)],
                        acc_sh.at[pl.ds(sid * ROWS_PS + t * DRAIN, DRAIN)])
    plsc.subcore_barrier()

    def _fire(j0, rows, sem):
        for b in range(GRP):
            pltpu.async_copy(
                y_hbm.at[src_v.at[j0 + b]], rows.at[pl.ds(b * KE, KE)], sem)

    def _drain_scatter(j0, rows, sem):
        for b in range(GRP):
            pltpu.make_async_copy(
                y_hbm.at[src_v.at[j0 + b]], rows.at[pl.ds(b * KE, KE)],
                sem).wait()
        for b in range(GRP):
            pltpu.sync_copy(rows.at[pl.ds(b * KE, KE)],
                            acc_sh.at[dst_v.at[j0 + b]], add=True)

    for seg in range(NSEG):
        pltpu.sync_copy(src_hbm.at[cid, sid, pl.ds(seg * SEG, SEG)], src_v)
        pltpu.sync_copy(dst_hbm.at[sid, pl.ds(seg * SEG, SEG)], dst_v)
        _fire(0, rows_a, sem_a)

        def _two_groups(i, c):
            j0 = 2 * GRP * i
            _fire(j0 + GRP, rows_b, sem_b)
            _drain_scatter(j0, rows_a, sem_a)

            @pl.when(j0 + 2 * GRP < SEG)
            def _():
                _fire(j0 + 2 * GRP, rows_a, sem_a)

            _drain_scatter(j0 + GRP, rows_b, sem_b)
            return c

        lax.fori_loop(0, SEG // (2 * GRP), _two_groups, None)

    plsc.subcore_barrier()
    for t in range(ROWS_PS // DRAIN):
        r0 = sid * ROWS_PS + t * DRAIN
        pltpu.sync_copy(acc_sh.at[pl.ds(r0, DRAIN)], rows_a.at[pl.ds(0, DRAIN)])
        pltpu.sync_copy(rows_a.at[pl.ds(0, DRAIN)],
                        part_hbm.at[cid, pl.ds(r0, DRAIN)])


_edge_call = pl.kernel(
    _edge_body,
    out_type=jax.ShapeDtypeStruct((NC, N_PAD, HF), jnp.float32),
    mesh=_mesh,
    scratch_types=[
        pltpu.VMEM((SEG, KE), jnp.int32),
        pltpu.VMEM((SEG, KE), jnp.int32),
        pltpu.VMEM((GRP * KE, HF), jnp.float32),
        pltpu.VMEM((GRP * KE, HF), jnp.float32),
        pltpu.VMEM_SHARED((N_PAD, HF), jnp.float32),
        pltpu.SemaphoreType.DMA,
        pltpu.SemaphoreType.DMA,
    ],
    compiler_params=pltpu.CompilerParams(use_tc_tiling_on_sc=False),
)


def _tc_first_body(x_ref, ie_ref, degp_ref, w_ref, ysp_ref, y_ref, dis_ref):
    deg = degp_ref[0, :, 0:1] + degp_ref[1, :, 0:1] + 1.0
    dis = jnp.broadcast_to(lax.rsqrt(deg), (BLK, FEAT))
    h = ie_ref[...] * (1.0 + x_ref[...])
    y = jnp.dot(h, w_ref[...], preferred_element_type=jnp.float32) * dis
    y_ref[...] = y
    ysp_ref[0] = y[:, :HF]
    ysp_ref[1] = y[:, HF:]
    dis_ref[...] = dis


_tc_first = pl.pallas_call(
    _tc_first_body,
    grid=(N_NODES // BLK,),
    in_specs=[
        pl.BlockSpec((BLK, FEAT), lambda i: (i, 0)),
        pl.BlockSpec((BLK, FEAT), lambda i: (i, 0)),
        pl.BlockSpec((NC, BLK, DEGW), lambda i: (0, i, 0)),
        pl.BlockSpec((FEAT, FEAT), lambda i: (0, 0)),
    ],
    out_specs=[
        pl.BlockSpec((NC, BLK, HF), lambda i: (0, i, 0)),
        pl.BlockSpec((BLK, FEAT), lambda i: (i, 0)),
        pl.BlockSpec((BLK, FEAT), lambda i: (i, 0)),
    ],
    out_shape=[
        jax.ShapeDtypeStruct((NC, N_NODES, HF), jnp.float32),
        jax.ShapeDtypeStruct((N_NODES, FEAT), jnp.float32),
        jax.ShapeDtypeStruct((N_NODES, FEAT), jnp.float32),
    ],
)


def _tc_mid_body(p_ref, y_ref, dis_ref, b_ref, w_ref, ysp_ref, y2_ref):
    agg = jnp.concatenate([p_ref[0], p_ref[1]], axis=-1) + y_ref[...]
    dis = dis_ref[...]
    o = agg * dis + b_ref[...]
    h = jnp.where(o >= 0, o, 0.01 * o)
    y2 = jnp.dot(h, w_ref[...], preferred_element_type=jnp.float32) * dis
    y2_ref[...] = y2
    ysp_ref[0] = y2[:, :HF]
    ysp_ref[1] = y2[:, HF:]


_tc_mid = pl.pallas_call(
    _tc_mid_body,
    grid=(N_NODES // BLK,),
    in_specs=[
        pl.BlockSpec((NC, BLK, HF), lambda i: (0, i, 0)),
        pl.BlockSpec((BLK, FEAT), lambda i: (i, 0)),
        pl.BlockSpec((BLK, FEAT), lambda i: (i, 0)),
        pl.BlockSpec((1, FEAT), lambda i: (0, 0)),
        pl.BlockSpec((FEAT, FEAT), lambda i: (0, 0)),
    ],
    out_specs=[
        pl.BlockSpec((NC, BLK, HF), lambda i: (0, i, 0)),
        pl.BlockSpec((BLK, FEAT), lambda i: (i, 0)),
    ],
    out_shape=[
        jax.ShapeDtypeStruct((NC, N_NODES, HF), jnp.float32),
        jax.ShapeDtypeStruct((N_NODES, FEAT), jnp.float32),
    ],
)


def _tc_last_body(p_ref, y_ref, dis_ref, b_ref, o_ref):
    agg = jnp.concatenate([p_ref[0], p_ref[1]], axis=-1) + y_ref[...]
    o = agg * dis_ref[...] + b_ref[...]
    o_ref[...] = jnp.where(o >= 0, o, 0.01 * o)


_tc_last = pl.pallas_call(
    _tc_last_body,
    grid=(N_NODES // BLK,),
    in_specs=[
        pl.BlockSpec((NC, BLK, HF), lambda i: (0, i, 0)),
        pl.BlockSpec((BLK, FEAT), lambda i: (i, 0)),
        pl.BlockSpec((BLK, FEAT), lambda i: (i, 0)),
        pl.BlockSpec((1, FEAT), lambda i: (0, 0)),
    ],
    out_specs=pl.BlockSpec((BLK, FEAT), lambda i: (i, 0)),
    out_shape=jax.ShapeDtypeStruct((N_NODES, FEAT), jnp.float32),
)


def kernel(x, edge_index, initial_embedding, W1, b1, W2, b2):
    src = edge_index[0].astype(jnp.int32)
    dst = edge_index[1].astype(jnp.int32)
    dst_d = dst.reshape(NW, NCHUNK_D, K)
    dst_e = dst.reshape(NS, NCHUNK_E, KE)
    src_e = (src.reshape(1, NS, NCHUNK_E, KE)
             + (jnp.arange(NC, dtype=jnp.int32) * N_NODES)[:, None, None, None])

    degp = _deg_call(dst_d)
    ysp0, y0, dis = _tc_first(x, initial_embedding, degp, W1)
    p0 = _edge_call(ysp0.reshape(NC * N_NODES, HF), src_e, dst_e)
    ysp1, y1 = _tc_mid(p0, y0, dis, b1.reshape(1, FEAT), W2)
    p1 = _edge_call(ysp1.reshape(NC * N_NODES, HF), src_e, dst_e)
    return _tc_last(p1, y1, dis, b2.reshape(1, FEAT))


# speedup vs baseline: 25.4624x; 1.0089x over previous
import jax
import jax.numpy as jnp
from jax import lax
from jax.experimental import pallas as pl
from jax.experimental.pallas import tpu as pltpu
from jax.experimental.pallas import tpu_sc as plsc

N_NODES = 10000
FEAT = 128
HF = FEAT // 2
N_EDGES = 320000

NC = 2
NS = 16
NW = NC * NS
K = 40
KE = 80
GRP = 5
NCHUNK_D = N_EDGES // NW // K
NCHUNK_E = N_EDGES // NS // KE
NSEG = 5
SEG = NCHUNK_E // NSEG
N_PAD = 10240
ROWS_PS = N_PAD // NS
DRAIN = 128
DEGW = 16
BLK = 1000

_mesh = plsc.VectorSubcoreMesh(
    core_axis_name="c", subcore_axis_name="s", num_cores=NC, num_subcores=NS
)


def _deg_body(dst_hbm, degp_hbm, dst_v, ones_v, buf_v, acc_sh, sem_d):
    cid = lax.axis_index("c")
    sid = lax.axis_index("s")
    wid = cid * NS + sid
    pltpu.sync_copy(dst_hbm.at[wid], dst_v)

    def _initrow(i, c):
        ones_v[i, :] = jnp.ones((16,), jnp.float32)
        return c

    lax.fori_loop(0, K, _initrow, None)

    def _zrow(i, c):
        buf_v[i, :] = jnp.zeros((16,), jnp.float32)
        return c

    lax.fori_loop(0, ROWS_PS, _zrow, None)
    pltpu.sync_copy(buf_v, acc_sh.at[pl.ds(sid * ROWS_PS, ROWS_PS)])
    plsc.subcore_barrier()

    def _grp(i, c):
        j0 = 10 * i
        for b in range(10):
            pltpu.async_copy(ones_v, acc_sh.at[dst_v.at[j0 + b]], sem_d,
                             add=True)
        for b in range(10):
            pltpu.make_async_copy(ones_v, acc_sh.at[dst_v.at[j0 + b]],
                                  sem_d).wait()
        return c

    lax.fori_loop(0, NCHUNK_D // 10, _grp, None)
    plsc.subcore_barrier()
    pltpu.sync_copy(acc_sh.at[pl.ds(sid * ROWS_PS, ROWS_PS)], buf_v)
    pltpu.sync_copy(buf_v, degp_hbm.at[cid, pl.ds(sid * ROWS_PS, ROWS_PS)])


_deg_call = pl.kernel(
    _deg_body,
    out_type=jax.ShapeDtypeStruct((NC, N_PAD, DEGW), jnp.float32),
    mesh=_mesh,
    scratch_types=[
        pltpu.VMEM((NCHUNK_D, K), jnp.int32),
        pltpu.VMEM((K, DEGW), jnp.float32),
        pltpu.VMEM((ROWS_PS, DEGW), jnp.float32),
        pltpu.VMEM_SHARED((N_PAD, DEGW), jnp.float32),
        pltpu.SemaphoreType.DMA,
    ],
    compiler_params=pltpu.CompilerParams(use_tc_tiling_on_sc=False),
)


def _edge_body(y_hbm, src_hbm, dst_hbm, part_hbm,
               src_v, dst_v, rows_a, rows_b, acc_sh,
               sem_a, sem_b, sem_sa, sem_sb):
    cid = lax.axis_index("c")
    sid = lax.axis_index("s")

    def _zrow(i, c):
        for t in range(HF // 16):
            rows_a[i, pl.ds(t * 16, 16)] = jnp.zeros((16,), jnp.float32)
        return c

    lax.fori_loop(0, DRAIN, _zrow, None)
    for t in range(ROWS_PS // DRAIN):
        pltpu.sync_copy(rows_a.at[pl.ds(0, DRAIN)],
                        acc_sh.at[pl.ds(sid * ROWS_PS + t * DRAIN, DRAIN)])
    plsc.subcore_barrier()

    def _fire(j0, rows, sem):
        for b in range(GRP):
            pltpu.async_copy(
                y_hbm.at[src_v.at[j0 + b]], rows.at[pl.ds(b * KE, KE)], sem)

    def _drain_scatter(j0, rows, sem, sem_s):
        for b in range(GRP):
            pltpu.make_async_copy(
                y_hbm.at[src_v.at[j0 + b]], rows.at[pl.ds(b * KE, KE)],
                sem).wait()
        for b in range(GRP):
            pltpu.async_copy(rows.at[pl.ds(b * KE, KE)],
                             acc_sh.at[dst_v.at[j0 + b]], sem_s, add=True)
        for b in range(GRP):
            pltpu.make_async_copy(rows.at[pl.ds(b * KE, KE)],
                                  acc_sh.at[dst_v.at[j0 + b]], sem_s).wait()

    for seg in range(NSEG):
        pltpu.sync_copy(src_hbm.at[cid, sid, pl.ds(seg * SEG, SEG)], src_v)
        pltpu.sync_copy(dst_hbm.at[sid, pl.ds(seg * SEG, SEG)], dst_v)
        _fire(0, rows_a, sem_a)

        def _two_groups(i, c):
            j0 = 2 * GRP * i
            _fire(j0 + GRP, rows_b, sem_b)
            _drain_scatter(j0, rows_a, sem_a, sem_sa)

            @pl.when(j0 + 2 * GRP < SEG)
            def _():
                _fire(j0 + 2 * GRP, rows_a, sem_a)

            _drain_scatter(j0 + GRP, rows_b, sem_b, sem_sb)
            return c

        lax.fori_loop(0, SEG // (2 * GRP), _two_groups, None)

    plsc.subcore_barrier()
    for t in range(ROWS_PS // DRAIN):
        r0 = sid * ROWS_PS + t * DRAIN
        pltpu.sync_copy(acc_sh.at[pl.ds(r0, DRAIN)], rows_a.at[pl.ds(0, DRAIN)])
        pltpu.sync_copy(rows_a.at[pl.ds(0, DRAIN)],
                        part_hbm.at[cid, pl.ds(r0, DRAIN)])


_edge_call = pl.kernel(
    _edge_body,
    out_type=jax.ShapeDtypeStruct((NC, N_PAD, HF), jnp.float32),
    mesh=_mesh,
    scratch_types=[
        pltpu.VMEM((SEG, KE), jnp.int32),
        pltpu.VMEM((SEG, KE), jnp.int32),
        pltpu.VMEM((GRP * KE, HF), jnp.float32),
        pltpu.VMEM((GRP * KE, HF), jnp.float32),
        pltpu.VMEM_SHARED((N_PAD, HF), jnp.float32),
        pltpu.SemaphoreType.DMA,
        pltpu.SemaphoreType.DMA,
        pltpu.SemaphoreType.DMA,
        pltpu.SemaphoreType.DMA,
    ],
    compiler_params=pltpu.CompilerParams(use_tc_tiling_on_sc=False),
)


def _tc_first_body(x_ref, ie_ref, degp_ref, w_ref, ysp_ref, y_ref, dis_ref):
    deg = degp_ref[0, :, 0:1] + degp_ref[1, :, 0:1] + 1.0
    dis = jnp.broadcast_to(lax.rsqrt(deg), (BLK, FEAT))
    h = ie_ref[...] * (1.0 + x_ref[...])
    y = jnp.dot(h, w_ref[...], preferred_element_type=jnp.float32) * dis
    y_ref[...] = y
    ysp_ref[0] = y[:, :HF]
    ysp_ref[1] = y[:, HF:]
    dis_ref[...] = dis


_tc_first = pl.pallas_call(
    _tc_first_body,
    grid=(N_NODES // BLK,),
    in_specs=[
        pl.BlockSpec((BLK, FEAT), lambda i: (i, 0)),
        pl.BlockSpec((BLK, FEAT), lambda i: (i, 0)),
        pl.BlockSpec((NC, BLK, DEGW), lambda i: (0, i, 0)),
        pl.BlockSpec((FEAT, FEAT), lambda i: (0, 0)),
    ],
    out_specs=[
        pl.BlockSpec((NC, BLK, HF), lambda i: (0, i, 0)),
        pl.BlockSpec((BLK, FEAT), lambda i: (i, 0)),
        pl.BlockSpec((BLK, FEAT), lambda i: (i, 0)),
    ],
    out_shape=[
        jax.ShapeDtypeStruct((NC, N_NODES, HF), jnp.float32),
        jax.ShapeDtypeStruct((N_NODES, FEAT), jnp.float32),
        jax.ShapeDtypeStruct((N_NODES, FEAT), jnp.float32),
    ],
)


def _tc_mid_body(p_ref, y_ref, dis_ref, b_ref, w_ref, ysp_ref, y2_ref):
    agg = jnp.concatenate([p_ref[0], p_ref[1]], axis=-1) + y_ref[...]
    dis = dis_ref[...]
    o = agg * dis + b_ref[...]
    h = jnp.where(o >= 0, o, 0.01 * o)
    y2 = jnp.dot(h, w_ref[...], preferred_element_type=jnp.float32) * dis
    y2_ref[...] = y2
    ysp_ref[0] = y2[:, :HF]
    ysp_ref[1] = y2[:, HF:]


_tc_mid = pl.pallas_call(
    _tc_mid_body,
    grid=(N_NODES // BLK,),
    in_specs=[
        pl.BlockSpec((NC, BLK, HF), lambda i: (0, i, 0)),
        pl.BlockSpec((BLK, FEAT), lambda i: (i, 0)),
        pl.BlockSpec((BLK, FEAT), lambda i: (i, 0)),
        pl.BlockSpec((1, FEAT), lambda i: (0, 0)),
        pl.BlockSpec((FEAT, FEAT), lambda i: (0, 0)),
    ],
    out_specs=[
        pl.BlockSpec((NC, BLK, HF), lambda i: (0, i, 0)),
        pl.BlockSpec((BLK, FEAT), lambda i: (i, 0)),
    ],
    out_shape=[
        jax.ShapeDtypeStruct((NC, N_NODES, HF), jnp.float32),
        jax.ShapeDtypeStruct((N_NODES, FEAT), jnp.float32),
    ],
)


def _tc_last_body(p_ref, y_ref, dis_ref, b_ref, o_ref):
    agg = jnp.concatenate([p_ref[0], p_ref[1]], axis=-1) + y_ref[...]
    o = agg * dis_ref[...] + b_ref[...]
    o_ref[...] = jnp.where(o >= 0, o, 0.01 * o)


_tc_last = pl.pallas_call(
    _tc_last_body,
    grid=(N_NODES // BLK,),
    in_specs=[
        pl.BlockSpec((NC, BLK, HF), lambda i: (0, i, 0)),
        pl.BlockSpec((BLK, FEAT), lambda i: (i, 0)),
        pl.BlockSpec((BLK, FEAT), lambda i: (i, 0)),
        pl.BlockSpec((1, FEAT), lambda i: (0, 0)),
    ],
    out_specs=pl.BlockSpec((BLK, FEAT), lambda i: (i, 0)),
    out_shape=jax.ShapeDtypeStruct((N_NODES, FEAT), jnp.float32),
)


def kernel(x, edge_index, initial_embedding, W1, b1, W2, b2):
    src = edge_index[0].astype(jnp.int32)
    dst = edge_index[1].astype(jnp.int32)
    dst_d = dst.reshape(NW, NCHUNK_D, K)
    dst_e = dst.reshape(NS, NCHUNK_E, KE)
    src_e = (src.reshape(1, NS, NCHUNK_E, KE)
             + (jnp.arange(NC, dtype=jnp.int32) * N_NODES)[:, None, None, None])

    degp = _deg_call(dst_d)
    ysp0, y0, dis = _tc_first(x, initial_embedding, degp, W1)
    p0 = _edge_call(ysp0.reshape(NC * N_NODES, HF), src_e, dst_e)
    ysp1, y1 = _tc_mid(p0, y0, dis, b1.reshape(1, FEAT), W2)
    p1 = _edge_call(ysp1.reshape(NC * N_NODES, HF), src_e, dst_e)
    return _tc_last(p1, y1, dis, b2.reshape(1, FEAT))
